# Initial kernel scaffold; baseline (speedup 1.0000x reference)
import jax
import jax.numpy as jnp
from jax.experimental import pallas as pl


def kernel(logits, labels, T, U):
    raise NotImplementedError("write your pallas kernel here")


# speedup vs baseline: 4.5999x; 4.5999x over previous
import jax
import jax.numpy as jnp
from jax.experimental import pallas as pl
from jax.experimental.pallas import tpu as pltpu

_B, _MAXT, _MAXU, _A = 8, 512, 65, 256
_TB = 16
_NT = _MAXT // _TB
_NEG = -1e30


def _shift_right(x, d, fill):
    n = x.shape[-1]
    pad = jnp.full(x.shape[:-1] + (d,), fill, x.dtype)
    return jnp.concatenate([pad, x[..., : n - d]], axis=-1)


def _cumsum_lanes(x):
    n = x.shape[-1]
    d = 1
    while d < n:
        x = x + _shift_right(x, d, 0.0)
        d *= 2
    return x


def _logaddexp(a, b):
    m = jnp.maximum(a, b)
    return m + jnp.log1p(jnp.exp(-jnp.abs(a - b)))


def _logcumsumexp_lanes(x):
    n = x.shape[-1]
    d = 1
    while d < n:
        x = _logaddexp(x, _shift_right(x, d, _NEG))
        d *= 2
    return x


def _fused_kernel(lab_ref, t_ref, u_ref, x_ref, out_ref, alpha_ref, pb_ref, acc_ref):
    i = pl.program_id(0)
    lab = lab_ref[...]
    aio = jax.lax.broadcasted_iota(jnp.int32, (_B, _MAXU - 1, _A), 2)
    onehot = lab[:, :, None] == aio
    uio = jax.lax.broadcasted_iota(jnp.int32, (_B, _MAXU), 1)
    umask = uio == u_ref[...]
    tlast = t_ref[...] - 1

    alpha = alpha_ref[...]
    pb = pb_ref[...]
    acc = jnp.where(i == 0, 0.0, acc_ref[...])

    prev_bl = pb
    for j in range(_TB):
        x = x_ref[:, j]
        m = jnp.max(x, axis=-1)
        lse = m + jnp.log(jnp.sum(jnp.exp(x - m[..., None]), axis=-1))
        bl = x[:, :, 0] - lse
        em = (
            jnp.sum(jnp.where(onehot, x[:, : _MAXU - 1, :], 0.0), axis=-1)
            - lse[:, : _MAXU - 1]
        )
        ecol = _shift_right(
            jnp.concatenate([em, jnp.zeros((_B, 1), jnp.float32)], axis=1), 1, 0.0
        )
        cum_e = _cumsum_lanes(ecol)

        t = i * _TB + j
        ne = alpha + prev_bl
        alpha_new = cum_e + _logcumsumexp_lanes(ne - cum_e)
        if j == 0:
            alpha = jnp.where(i == 0, cum_e, alpha_new)
        else:
            alpha = alpha_new

        acc = acc + jnp.where((tlast == t) & umask, alpha + bl, 0.0)
        prev_bl = bl

    alpha_ref[...] = alpha
    pb_ref[...] = prev_bl
    acc_ref[...] = acc
    out_ref[...] = -jnp.sum(acc, axis=(0, 1), keepdims=True) / _B


def kernel(logits, labels, T, U):
    lab = labels.astype(jnp.int32)
    tv = T.astype(jnp.int32).reshape(_B, 1)
    uv = U.astype(jnp.int32).reshape(_B, 1)
    out = pl.pallas_call(
        _fused_kernel,
        grid=(_NT,),
        in_specs=[
            pl.BlockSpec((_B, _MAXU - 1), lambda i: (0, 0)),
            pl.BlockSpec((_B, 1), lambda i: (0, 0)),
            pl.BlockSpec((_B, 1), lambda i: (0, 0)),
            pl.BlockSpec((_B, _TB, _MAXU, _A), lambda i: (0, i, 0, 0)),
        ],
        out_specs=pl.BlockSpec((1, 1), lambda i: (0, 0)),
        out_shape=jax.ShapeDtypeStruct((1, 1), jnp.float32),
        scratch_shapes=[
            pltpu.VMEM((_B, _MAXU), jnp.float32),
            pltpu.VMEM((_B, _MAXU), jnp.float32),
            pltpu.VMEM((_B, _MAXU), jnp.float32),
        ],
    )(lab, tv, uv, logits)
    return out[0, 0]
